# baseline (speedup 1.0000x reference)
import jax
import jax.numpy as jnp
from jax import lax
from jax.experimental import pallas as pl
from jax.experimental.pallas import tpu as pltpu
from jax.experimental.pallas import tpu_sc as plsc

D = 64
B, L = 64, 2048
N = B * L
N_T, N_B, N_P, N_K, N_D = 64, 128, 128, 256, 128

NC, NS = 2, 16
NW = NC * NS
TPW = N // NW
IG = 128
CHUNK = 128
NCHUNK = TPW // CHUNK


def _project_body(tt, bt, pt, kt, dt, w, b, otb, opd, otok):
    bias = b[0, :]
    p_t = jnp.dot(tt[...], w[0:64, :], preferred_element_type=jnp.float32) + bias
    p_b = jnp.dot(bt[...], w[64:128, :], preferred_element_type=jnp.float32)
    p_p = jnp.dot(pt[...], w[128:192, :], preferred_element_type=jnp.float32)
    p_k = jnp.dot(kt[...], w[192:256, :], preferred_element_type=jnp.float32)
    p_d = jnp.dot(dt[...], w[256:320, :], preferred_element_type=jnp.float32)
    for i in range(N_T):
        otb[pl.ds(i * N_B, N_B), :] = p_b + p_t[i:i + 1, :]
    for i in range(N_P):
        opd[pl.ds(i * N_D, N_D), :] = p_d + p_p[i:i + 1, :]
    otok[...] = p_k


def _project(tt, bt, pt, kt, dt, w, b):
    return pl.pallas_call(
        _project_body,
        out_shape=[
            jax.ShapeDtypeStruct((N_T * N_B, D), jnp.float32),
            jax.ShapeDtypeStruct((N_P * N_D, D), jnp.float32),
            jax.ShapeDtypeStruct((N_K, D), jnp.float32),
        ],
    )(tt, bt, pt, kt, dt, w, b.reshape(1, D))


def _lookup_body(ttb, tpd, ttok, iall, out,
                 vi0, vi1, ci0, ci1, rg0, rg1, rb0, rb1, rc0, rc1,
                 ac0, ac1, sidx, sg, so):
    cid = lax.axis_index("c")
    sid = lax.axis_index("s")
    wid = sid * NC + cid
    row0 = wid * NCHUNK
    orow0 = wid * (TPW // 2)
    vi = (vi0, vi1)
    ci = (ci0, ci1)
    rg = (rg0, rg1)
    rb = (rb0, rb1)
    rc = (rc0, rc1)
    ac = (ac0, ac1)

    def idx_cp(k):
        return pltpu.make_async_copy(
            iall.at[pl.ds(row0 + k, 1)], vi[k & 1], sidx)

    def gather_cps(k):
        p = k & 1
        return [
            pltpu.make_async_copy(ttb.at[ci[p].at[0, 0]], rg[p], sg),
            pltpu.make_async_copy(tpd.at[ci[p].at[0, 1]], rb[p], sg),
            pltpu.make_async_copy(ttok.at[vi[p].at[0, 3]], rc[p], sg),
        ]

    def out_cp(k):
        return pltpu.make_async_copy(
            ac[k & 1], out.at[pl.ds(orow0 + k * (CHUNK // 2), CHUNK // 2)], so)

    def combine(k):
        p = k & 1
        vip, cip = vi[p], ci[p]

        def cb(j, c):
            sl = pl.ds(j * 16, 16)
            cip[0, 0, sl] = vip[0, 0, sl] * N_B + vip[0, 1, sl]
            cip[0, 1, sl] = vip[0, 2, sl] * N_D + vip[0, 4, sl]
            return c
        lax.fori_loop(0, IG // 16, cb, 0)

    def accum(k):
        p = k & 1
        rgp, rbp, rcp, acp = rg[p], rb[p], rc[p], ac[p]

        def ab(q, c):
            for h in range(2):
                for cc in range(D // 16):
                    src = pl.ds(cc * 16, 16)
                    dst = pl.ds(h * D + cc * 16, 16)
                    acp[q, dst] = (rgp[2 * q + h, src] + rbp[2 * q + h, src]
                                   + rcp[2 * q + h, src])
            return c
        lax.fori_loop(0, CHUNK // 2, ab, 0)

    idx_cp(0).start()
    idx_cp(0).wait()
    combine(0)
    for cp in gather_cps(0):
        cp.start()
    if NCHUNK > 1:
        idx_cp(1).start()

    for k in range(NCHUNK):
        if k + 1 < NCHUNK:
            idx_cp(k + 1).wait()
            combine(k + 1)
        for cp in gather_cps(k):
            cp.wait()
        if k >= 1:
            out_cp(k - 1).wait()
        if k + 1 < NCHUNK:
            for cp in gather_cps(k + 1):
                cp.start()
            if k + 2 < NCHUNK:
                idx_cp(k + 2).start()
        accum(k)
        out_cp(k).start()
    out_cp(NCHUNK - 1).wait()


def _lookup(ttb, tpd, ttok, iall):
    mesh = plsc.VectorSubcoreMesh(core_axis_name="c", subcore_axis_name="s")
    f = pl.kernel(
        _lookup_body,
        out_type=jax.ShapeDtypeStruct((N // 2, 2 * D), jnp.float32),
        mesh=mesh,
        scratch_types=[
            pltpu.VMEM((1, 5, IG), jnp.int32),
            pltpu.VMEM((1, 5, IG), jnp.int32),
            pltpu.VMEM((1, 2, IG), jnp.int32),
            pltpu.VMEM((1, 2, IG), jnp.int32),
            pltpu.VMEM((CHUNK, D), jnp.float32),
            pltpu.VMEM((CHUNK, D), jnp.float32),
            pltpu.VMEM((CHUNK, D), jnp.float32),
            pltpu.VMEM((CHUNK, D), jnp.float32),
            pltpu.VMEM((CHUNK, D), jnp.float32),
            pltpu.VMEM((CHUNK, D), jnp.float32),
            pltpu.VMEM((CHUNK // 2, 2 * D), jnp.float32),
            pltpu.VMEM((CHUNK // 2, 2 * D), jnp.float32),
            pltpu.SemaphoreType.DMA,
            pltpu.SemaphoreType.DMA,
            pltpu.SemaphoreType.DMA,
        ],
        compiler_params=pltpu.CompilerParams(use_tc_tiling_on_sc=False),
    )
    return f(ttb, tpd, ttok, iall)


def kernel(Tempo, Bar, Position, Token, Duration, tempo_table, bar_table,
           pos_table, token_table, dur_table, W_dec, b_dec):
    ttb, tpd, ttok = _project(tempo_table, bar_table, pos_table,
                              token_table, dur_table, W_dec, b_dec)
    iall = (jnp.stack([Tempo.reshape(N), Bar.reshape(N), Position.reshape(N),
                       Token.reshape(N), Duration.reshape(N)])
            .reshape(5, N // IG, IG).transpose(1, 0, 2))
    out = _lookup(ttb, tpd, ttok, iall)
    return out.reshape(B, L, D)
